# Initial kernel scaffold; baseline (speedup 1.0000x reference)
import jax
import jax.numpy as jnp
from jax.experimental import pallas as pl


def kernel(input_ids, position_ids, word_table, pos_table):
    raise NotImplementedError("write your pallas kernel here")


# speedup vs baseline: 1.3764x; 1.3764x over previous
import functools

import jax
import jax.numpy as jnp
from jax import lax
from jax.experimental import pallas as pl
from jax.experimental.pallas import tpu as pltpu
from jax.experimental.pallas import tpu_sc as plsc

NC = 2
NS = 16
NW = NC * NS
LANES = 16

H = 1024
K = 16


def _emb_body(CH, widx, pidx, word, pos, out, idw, idp, bw, bp, sem_w, sem_p):
    c = lax.axis_index("c")
    s = lax.axis_index("s")
    wid = s * NC + c

    pltpu.sync_copy(widx.at[wid], idw)
    pltpu.sync_copy(pidx.at[wid], idp)

    @pl.loop(0, CH)
    def chunk(g):
        cw = pltpu.async_copy(word.at[idw.at[g]], bw, sem_w)
        cp = pltpu.async_copy(pos.at[idp.at[g]], bp, sem_p)
        cw.wait()
        cp.wait()

        @pl.loop(0, K * H // LANES, unroll=8)
        def add(i):
            r = i // (H // LANES)
            col = (i % (H // LANES)) * LANES
            bw[r, pl.ds(col, LANES)] = (
                bw[r, pl.ds(col, LANES)] + bp[r, pl.ds(col, LANES)]
            )

        pltpu.sync_copy(bw, out.at[wid, g])


def kernel(input_ids, position_ids, word_table, pos_table):
    B, S = input_ids.shape
    N = B * S
    T = N // NW
    CH = T // K

    widx = input_ids.reshape(NW, CH, K).astype(jnp.int32)
    pidx = position_ids.reshape(NW, CH, K).astype(jnp.int32)

    mesh = plsc.VectorSubcoreMesh(core_axis_name="c", subcore_axis_name="s")
    out = pl.kernel(
        functools.partial(_emb_body, CH),
        out_type=jax.ShapeDtypeStruct((NW, CH, K, H), jnp.float32),
        mesh=mesh,
        scratch_types=[
            pltpu.VMEM((CH, K), jnp.int32),
            pltpu.VMEM((CH, K), jnp.int32),
            pltpu.VMEM((K, H), jnp.float32),
            pltpu.VMEM((K, H), jnp.float32),
            pltpu.SemaphoreType.DMA,
            pltpu.SemaphoreType.DMA,
        ],
    )(widx, pidx, word_table, pos_table)
    return out.reshape(B, S, H)


# speedup vs baseline: 2.2748x; 1.6527x over previous
import functools

import jax
import jax.numpy as jnp
from jax import lax
from jax.experimental import pallas as pl
from jax.experimental.pallas import tpu as pltpu
from jax.experimental.pallas import tpu_sc as plsc

NC = 2
NS = 16
NW = NC * NS
LANES = 16

H = 1024
K = 16
NBUF = 2


def _emb_body(CH, widx, pidx, word, pos, out,
              idw, idp, bw, bp, bo, sems):
    c = lax.axis_index("c")
    s = lax.axis_index("s")
    wid = s * NC + c

    sem_w = sems[0:NBUF]
    sem_p = sems[NBUF:2 * NBUF]
    sem_o = sems[2 * NBUF:3 * NBUF]

    pltpu.sync_copy(widx.at[wid], idw)
    pltpu.sync_copy(pidx.at[wid], idp)

    def issue_gathers(g, b):
        pltpu.async_copy(word.at[idw.at[g]], bw.at[b], sem_w[b])
        pltpu.async_copy(pos.at[idp.at[g]], bp.at[b], sem_p[b])

    def wait_gathers(g, b):
        pltpu.make_async_copy(word.at[idw.at[g]], bw.at[b], sem_w[b]).wait()
        pltpu.make_async_copy(pos.at[idp.at[g]], bp.at[b], sem_p[b]).wait()

    def wait_out(g, b):
        pltpu.make_async_copy(bo.at[b], out.at[wid, g], sem_o[b]).wait()

    for b in range(NBUF):
        issue_gathers(b, b)

    @pl.loop(0, CH // NBUF)
    def superstep(t):
        for b in range(NBUF):
            g = t * NBUF + b
            wait_gathers(g, b)

            @pl.when(t >= 1)
            def _():
                wait_out(g, b)

            @pl.loop(0, K * H // LANES, unroll=8)
            def add(i):
                r = i // (H // LANES)
                col = (i % (H // LANES)) * LANES
                bo[b, r, pl.ds(col, LANES)] = (
                    bw[b, r, pl.ds(col, LANES)] + bp[b, r, pl.ds(col, LANES)]
                )

            @pl.when(t < CH // NBUF - 1)
            def _():
                issue_gathers(g + NBUF, b)

            pltpu.async_copy(bo.at[b], out.at[wid, g], sem_o[b])

    for b in range(NBUF):
        wait_out(CH - NBUF + b, b)


def kernel(input_ids, position_ids, word_table, pos_table):
    B, S = input_ids.shape
    N = B * S
    T = N // NW
    CH = T // K

    widx = input_ids.reshape(NW, CH, K).astype(jnp.int32)
    pidx = position_ids.reshape(NW, CH, K).astype(jnp.int32)

    mesh = plsc.VectorSubcoreMesh(core_axis_name="c", subcore_axis_name="s")
    out = pl.kernel(
        functools.partial(_emb_body, CH),
        out_type=jax.ShapeDtypeStruct((NW, CH, K, H), jnp.float32),
        mesh=mesh,
        scratch_types=[
            pltpu.VMEM((CH, K), jnp.int32),
            pltpu.VMEM((CH, K), jnp.int32),
            pltpu.VMEM((NBUF, K, H), jnp.float32),
            pltpu.VMEM((NBUF, K, H), jnp.float32),
            pltpu.VMEM((NBUF, K, H), jnp.float32),
            [pltpu.SemaphoreType.DMA] * (3 * NBUF),
        ],
    )(widx, pidx, word_table, pos_table)
    return out.reshape(B, S, H)
